# baseline (speedup 1.0000x reference)
import functools

import jax
import jax.numpy as jnp
from jax import lax
from jax.experimental import pallas as pl
from jax.experimental.pallas import tpu as pltpu
from jax.experimental.pallas import tpu_sc as plsc

VOCAB = 1000000
DIM = 64
MAXLEN = 200
BATCH = 1024

NC = 2
NS = 16
NW = NC * NS
ROWS = BATCH * MAXLEN
RPW = ROWS // NW
CHUNK = 128
NCHUNK = RPW // CHUNK
LANES = 16


def _pe_table():
    rows = jnp.arange(MAXLEN, dtype=jnp.float32)[:, None]
    cols = jnp.arange(DIM // 2, dtype=jnp.float32)[None, :]
    denom = jnp.power(10000.0, 2.0 * cols / DIM)
    ang = rows / denom
    pe = jnp.zeros((MAXLEN, DIM), dtype=jnp.float32)
    pe = pe.at[:, 0::2].set(jnp.sin(ang))
    pe = pe.at[:, 1::2].set(jnp.cos(ang))
    return pe


_MESH = plsc.VectorSubcoreMesh(
    core_axis_name="c", subcore_axis_name="s", num_cores=NC, num_subcores=NS
)


@functools.partial(
    pl.kernel,
    out_type=jax.ShapeDtypeStruct((ROWS, DIM), jnp.float32),
    mesh=_MESH,
    scratch_types=[
        pltpu.VMEM((NCHUNK, CHUNK), jnp.int32),
        pltpu.VMEM((MAXLEN, DIM), jnp.float32),
        pltpu.VMEM((CHUNK, DIM), jnp.float32),
        pltpu.SemaphoreType.DMA,
    ],
    compiler_params=pltpu.CompilerParams(use_tc_tiling_on_sc=False),
)
def _emb_kernel(x_hbm, pe_hbm, table_hbm, out_hbm, idx_v, pe_v, rows_v, sem):
    wid = lax.axis_index("s") * NC + lax.axis_index("c")
    pltpu.sync_copy(x_hbm.at[wid], idx_v)
    pltpu.sync_copy(pe_hbm, pe_v)

    @pl.loop(0, NCHUNK)
    def _chunk(j):
        pltpu.async_copy(table_hbm.at[idx_v.at[j]], rows_v, sem).wait()
        gbase = lax.rem(j * CHUNK, MAXLEN)

        @pl.loop(0, CHUNK)
        def _row(r):
            p = lax.rem(gbase + r, MAXLEN)
            for c in range(DIM // LANES):
                sl = pl.ds(c * LANES, LANES)
                rows_v[r, sl] = rows_v[r, sl] + pe_v[p, sl]

        pltpu.sync_copy(
            rows_v, out_hbm.at[pl.ds(wid * RPW + j * CHUNK, CHUNK)]
        )


def kernel(x, table):
    pe = _pe_table()
    xf = x.reshape(NW, NCHUNK, CHUNK)
    out = _emb_kernel(xf, pe, table)
    return out.reshape(BATCH, MAXLEN, DIM)


# speedup vs baseline: 1.1758x; 1.1758x over previous
import functools

import jax
import jax.numpy as jnp
from jax import lax
from jax.experimental import pallas as pl
from jax.experimental.pallas import tpu as pltpu
from jax.experimental.pallas import tpu_sc as plsc

VOCAB = 1000000
DIM = 64
MAXLEN = 200
BATCH = 1024

NC = 2
NS = 16
NW = NC * NS
ROWS = BATCH * MAXLEN
RPW = ROWS // NW
CHUNK = 100
NCHUNK = RPW // CHUNK
NB = 4
LANES = 16


def _pe_table():
    rows = jnp.arange(MAXLEN, dtype=jnp.float32)[:, None]
    cols = jnp.arange(DIM // 2, dtype=jnp.float32)[None, :]
    denom = jnp.power(10000.0, 2.0 * cols / DIM)
    ang = rows / denom
    pe = jnp.zeros((MAXLEN, DIM), dtype=jnp.float32)
    pe = pe.at[:, 0::2].set(jnp.sin(ang))
    pe = pe.at[:, 1::2].set(jnp.cos(ang))
    return pe


_MESH = plsc.VectorSubcoreMesh(
    core_axis_name="c", subcore_axis_name="s", num_cores=NC, num_subcores=NS
)


@functools.partial(
    pl.kernel,
    out_type=jax.ShapeDtypeStruct((ROWS, DIM), jnp.float32),
    mesh=_MESH,
    scratch_types=[
        pltpu.VMEM((NCHUNK, CHUNK), jnp.int32),
        pltpu.VMEM((MAXLEN, DIM), jnp.float32),
    ]
    + [pltpu.VMEM((CHUNK, DIM), jnp.float32) for _ in range(NB)]
    + [pltpu.SemaphoreType.DMA for _ in range(2 * NB)],
    compiler_params=pltpu.CompilerParams(use_tc_tiling_on_sc=False),
)
def _emb_kernel(x_hbm, pe_hbm, table_hbm, out_hbm, idx_v, pe_v, *bufs_sems):
    rows = bufs_sems[:NB]
    gsem = bufs_sems[NB:2 * NB]
    ssem = bufs_sems[2 * NB:]
    wid = lax.axis_index("s") * NC + lax.axis_index("c")
    base = wid * RPW

    pltpu.sync_copy(x_hbm.at[wid], idx_v)
    pltpu.sync_copy(pe_hbm, pe_v)

    def fire_gather(j, b):
        pltpu.async_copy(table_hbm.at[idx_v.at[j]], rows[b], gsem[b])

    def wait_gather(j, b):
        pltpu.make_async_copy(table_hbm.at[idx_v.at[j]], rows[b], gsem[b]).wait()

    def out_slice(j):
        return out_hbm.at[pl.ds(base + j * CHUNK, CHUNK)]

    def fire_scatter(j, b):
        pltpu.async_copy(rows[b], out_slice(j), ssem[b])

    def wait_scatter(j, b):
        pltpu.make_async_copy(rows[b], out_slice(j), ssem[b]).wait()

    def add_pe(j, b):
        pbase = lax.rem(j, 2) * CHUNK

        @plsc.parallel_loop(0, CHUNK, unroll=4)
        def _row(r):
            p = pbase + r
            for c in range(DIM // LANES):
                sl = pl.ds(c * LANES, LANES)
                plsc.addupdate(rows[b].at[r, sl], pe_v[p, sl])

    fire_gather(0, 0)
    fire_gather(1, 1)
    for j in range(2):
        wait_gather(j, j)
        add_pe(j, j)
        fire_scatter(j, j)
        fire_gather(j + 2, j + 2)

    @pl.loop(0, (NCHUNK - 4) // NB)
    def _grp(g):
        for b4 in range(NB):
            j = 2 + g * NB + b4
            b = (2 + b4) % NB
            wait_gather(j, b)
            add_pe(j, b)
            fire_scatter(j, b)
            wait_scatter(j - 2, b4)
            fire_gather(j + 2, b4)

    for j in range(NCHUNK - 2, NCHUNK):
        b = j % NB
        wait_gather(j, b)
        add_pe(j, b)
        fire_scatter(j, b)
    for j in range(NCHUNK - NB, NCHUNK):
        wait_scatter(j, j % NB)


def kernel(x, table):
    pe = _pe_table()
    xf = x.reshape(NW, NCHUNK, CHUNK)
    out = _emb_kernel(xf, pe, table)
    return out.reshape(BATCH, MAXLEN, DIM)


# speedup vs baseline: 1.2501x; 1.0632x over previous
import functools

import jax
import jax.numpy as jnp
from jax import lax
from jax.experimental import pallas as pl
from jax.experimental.pallas import tpu as pltpu
from jax.experimental.pallas import tpu_sc as plsc

VOCAB = 1000000
DIM = 64
MAXLEN = 200
BATCH = 1024

NC = 2
NS = 16
NW = NC * NS
ROWS = BATCH * MAXLEN
RPW = ROWS // NW
CHUNK = 128
NCHUNK = RPW // CHUNK
NB = 2
LANES = 16
PEW = MAXLEN * DIM


def _pe_table():
    rows = jnp.arange(MAXLEN, dtype=jnp.float32)[:, None]
    cols = jnp.arange(DIM // 2, dtype=jnp.float32)[None, :]
    denom = jnp.power(10000.0, 2.0 * cols / DIM)
    ang = rows / denom
    pe = jnp.zeros((MAXLEN, DIM), dtype=jnp.float32)
    pe = pe.at[:, 0::2].set(jnp.sin(ang))
    pe = pe.at[:, 1::2].set(jnp.cos(ang))
    return pe


_MESH = plsc.VectorSubcoreMesh(
    core_axis_name="c", subcore_axis_name="s", num_cores=NC, num_subcores=NS
)


@functools.partial(
    pl.kernel,
    out_type=jax.ShapeDtypeStruct((ROWS, DIM), jnp.float32),
    mesh=_MESH,
    scratch_types=[
        pltpu.VMEM((RPW,), jnp.int32),
        pltpu.VMEM((RPW,), jnp.int32),
        pltpu.VMEM((RPW + LANES,), jnp.int32),
        pltpu.VMEM((PEW,), jnp.float32),
    ]
    + [pltpu.VMEM((CHUNK, 2 * DIM), jnp.float32) for _ in range(NB)]
    + [pltpu.VMEM((CHUNK, DIM), jnp.float32) for _ in range(NB)]
    + [pltpu.SemaphoreType.DMA for _ in range(2 * NB)],
    compiler_params=pltpu.CompilerParams(use_tc_tiling_on_sc=True),
)
def _emb_kernel(x_hbm, pe_hbm, tbl_hbm, out_hbm, idx_v, pair_v, hoff_v, pe_v,
                *bufs_sems):
    rows = bufs_sems[:NB]
    outs = bufs_sems[NB:2 * NB]
    gsem = bufs_sems[2 * NB:3 * NB]
    ssem = bufs_sems[3 * NB:]
    wid = lax.axis_index("s") * NC + lax.axis_index("c")
    base = wid * RPW

    pltpu.sync_copy(x_hbm.at[pl.ds(base, RPW)], idx_v)
    pltpu.sync_copy(pe_hbm, pe_v)

    @plsc.parallel_loop(0, RPW // LANES, unroll=4)
    def _pre(k):
        sl = pl.ds(k * LANES, LANES)
        v = idx_v[sl]
        pair_v[sl] = lax.shift_right_logical(v, 1)
        hoff_v[sl] = lax.shift_left(v & 1, 6)

    def fire_gather(j, b):
        pltpu.async_copy(
            tbl_hbm.at[pair_v.at[pl.ds(j * CHUNK, CHUNK)]], rows[b], gsem[b]
        )

    def wait_gather(j, b):
        pltpu.make_async_copy(
            tbl_hbm.at[pair_v.at[pl.ds(j * CHUNK, CHUNK)]], rows[b], gsem[b]
        ).wait()

    def out_slice(j):
        return out_hbm.at[pl.ds(base + j * CHUNK, CHUNK)]

    def fire_scatter(j, b):
        pltpu.async_copy(outs[b], out_slice(j), ssem[b])

    def wait_scatter(j, b):
        pltpu.make_async_copy(outs[b], out_slice(j), ssem[b]).wait()

    def select_add(j, b):
        jbase = j * CHUNK
        gbase = lax.rem(jbase, MAXLEN)

        @plsc.parallel_loop(0, CHUNK, unroll=2)
        def _row(r):
            h = hoff_v[pl.ds(jbase + r, LANES)][0]
            p = lax.rem(gbase + r, MAXLEN) * DIM
            for c in range(DIM // LANES):
                co = c * LANES
                outs[b][r, pl.ds(co, LANES)] = (
                    rows[b][r, pl.ds(h + co, LANES)]
                    + pe_v[pl.ds(p + co, LANES)]
                )

    fire_gather(0, 0)
    fire_gather(1, 1)
    for j in range(NB):
        wait_gather(j, j)
        select_add(j, j)
        fire_scatter(j, j)
        fire_gather(j + NB, j)

    @pl.loop(0, (NCHUNK - 2 * NB) // NB)
    def _grp(g):
        for b in range(NB):
            j = NB + g * NB + b
            wait_gather(j, b)
            wait_scatter(j - NB, b)
            select_add(j, b)
            fire_scatter(j, b)
            fire_gather(j + NB, b)

    for j in range(NCHUNK - NB, NCHUNK):
        b = j % NB
        wait_gather(j, b)
        wait_scatter(j - NB, b)
        select_add(j, b)
        fire_scatter(j, b)
    for j in range(NCHUNK - NB, NCHUNK):
        wait_scatter(j, j % NB)


def kernel(x, table):
    pe = _pe_table().reshape(-1)
    xf = x.reshape(-1)
    tbl = table.reshape(VOCAB // 2, 2 * DIM)
    out = _emb_kernel(xf, pe, tbl)
    return out.reshape(BATCH, MAXLEN, DIM)
